# baseline (speedup 1.0000x reference)
import jax
import jax.numpy as jnp
from jax import lax
from jax.experimental import pallas as pl
from jax.experimental.pallas import tpu as pltpu
from jax.experimental.pallas import tpu_sc as plsc

B = 16
C = 384
H = 96
W = 96
K = 90
L = 16
NC = 2
NS = 16
KPT = 3
NT = K // KPT


def _body(table, vert, out, vert_v, idx_v, gbuf, sem):
    wid = lax.axis_index("s") * NC + lax.axis_index("c")

    @pl.when(wid < NT)
    def _():
        pltpu.sync_copy(vert, vert_v)

        b_vec = lax.iota(jnp.int32, L)
        zero = jnp.zeros((L,), jnp.int32)
        one = zero + 1
        row0 = b_vec * (H * W)
        for kk in range(KPT):
            k_vec = zero + (KPT * wid + kk)
            i = plsc.load_gather(vert_v, [b_vec, zero, k_vec])
            j = plsc.load_gather(vert_v, [b_vec, one, k_vec])
            idx_v[pl.ds(kk * L, L)] = row0 + i * W + j

        pltpu.async_copy(table.at[idx_v], gbuf, sem).wait()
        pltpu.sync_copy(gbuf, out.at[wid])


@jax.jit
def _sampler(table, vert_flat):
    mesh = plsc.VectorSubcoreMesh(
        core_axis_name="c", subcore_axis_name="s", num_cores=NC, num_subcores=NS
    )
    f = pl.kernel(
        _body,
        out_type=jax.ShapeDtypeStruct((NT, KPT * B, C), jnp.float32),
        mesh=mesh,
        compiler_params=pltpu.CompilerParams(
            needs_layout_passes=False, use_tc_tiling_on_sc=True
        ),
        scratch_types=[
            pltpu.VMEM((B, 2, K), jnp.int32),
            pltpu.VMEM((KPT * L,), jnp.int32),
            pltpu.VMEM((KPT * L, C), jnp.float32),
            pltpu.SemaphoreType.DMA,
        ],
    )
    return f(table, vert_flat)


def kernel(x, vertexs):
    table = x.transpose(0, 2, 3, 1).reshape(B * H * W, C)
    vert_t = vertexs.astype(jnp.int32).transpose(0, 2, 1)
    out = _sampler(table, vert_t)
    return out.reshape(K, B, C).transpose(1, 2, 0)


# speedup vs baseline: 1.0015x; 1.0015x over previous
import jax
import jax.numpy as jnp
from jax import lax
from jax.experimental import pallas as pl
from jax.experimental.pallas import tpu as pltpu
from jax.experimental.pallas import tpu_sc as plsc

B = 16
C = 384
H = 96
W = 96
K = 90
L = 16
NC = 2
NS = 16
KPT = 3
NT = K // KPT


def _body(table, vert, out, vert_v, idx_v, gbuf, gsems, wsem):
    wid = lax.axis_index("s") * NC + lax.axis_index("c")

    @pl.when(wid < NT)
    def _():
        pltpu.sync_copy(vert, vert_v)

        b_vec = lax.iota(jnp.int32, L)
        zero = jnp.zeros((L,), jnp.int32)
        one = zero + 1
        row0 = b_vec * (H * W)
        for kk in range(KPT):
            k_vec = zero + (KPT * wid + kk)
            i = plsc.load_gather(vert_v, [b_vec, zero, k_vec])
            j = plsc.load_gather(vert_v, [b_vec, one, k_vec])
            idx_v[kk] = row0 + i * W + j

        gcps = [
            pltpu.async_copy(table.at[idx_v.at[kk]], gbuf.at[kk], gsems[kk])
            for kk in range(KPT)
        ]
        wcps = []
        for kk in range(KPT):
            gcps[kk].wait()
            wcps.append(
                pltpu.async_copy(
                    gbuf.at[kk], out.at[wid, pl.ds(kk * L, L)], wsem
                )
            )
        for wcp in wcps:
            wcp.wait()


@jax.jit
def _sampler(table, vert_flat):
    mesh = plsc.VectorSubcoreMesh(
        core_axis_name="c", subcore_axis_name="s", num_cores=NC, num_subcores=NS
    )
    f = pl.kernel(
        _body,
        out_type=jax.ShapeDtypeStruct((NT, KPT * B, C), jnp.float32),
        mesh=mesh,
        compiler_params=pltpu.CompilerParams(
            needs_layout_passes=False, use_tc_tiling_on_sc=True
        ),
        scratch_types=[
            pltpu.VMEM((B, 2, K), jnp.int32),
            pltpu.VMEM((KPT, L), jnp.int32),
            pltpu.VMEM((KPT, L, C), jnp.float32),
            [pltpu.SemaphoreType.DMA] * KPT,
            pltpu.SemaphoreType.DMA,
        ],
    )
    return f(table, vert_flat)


def kernel(x, vertexs):
    table = x.transpose(0, 2, 3, 1).reshape(B * H * W, C)
    vert_t = vertexs.astype(jnp.int32).transpose(0, 2, 1)
    out = _sampler(table, vert_t)
    return out.reshape(K, B, C).transpose(1, 2, 0)


# speedup vs baseline: 1.0073x; 1.0058x over previous
import jax
import jax.numpy as jnp
from jax import lax
from jax.experimental import pallas as pl
from jax.experimental.pallas import tpu as pltpu
from jax.experimental.pallas import tpu_sc as plsc

B = 16
C = 384
H = 96
W = 96
K = 90
L = 16
NC = 2
NS = 16
KPT = 3
NT = K // KPT


def _body(table, vert, out, vert_v, gbuf, gsems, wsem):
    wid = lax.axis_index("s") * NC + lax.axis_index("c")

    @pl.when(wid < NT)
    def _():
        pltpu.sync_copy(vert, vert_v)

        b_vec = lax.iota(jnp.int32, L)
        zero = jnp.zeros((L,), jnp.int32)
        one = zero + 1
        row0 = b_vec * (H * W)
        gcps = []
        for kk in range(KPT):
            k_vec = zero + (KPT * wid + kk)
            i = plsc.load_gather(vert_v, [b_vec, zero, k_vec])
            j = plsc.load_gather(vert_v, [b_vec, one, k_vec])
            idx_vec = row0 + i * W + j
            gcps.append(
                pltpu.async_copy(table.at[idx_vec], gbuf.at[kk], gsems[kk])
            )
        wcps = []
        for kk in range(KPT):
            gcps[kk].wait()
            wcps.append(
                pltpu.async_copy(
                    gbuf.at[kk], out.at[wid, pl.ds(kk * L, L)], wsem
                )
            )
        for wcp in wcps:
            wcp.wait()


@jax.jit
def _sampler(table, vert_flat):
    mesh = plsc.VectorSubcoreMesh(
        core_axis_name="c", subcore_axis_name="s", num_cores=NC, num_subcores=NS
    )
    f = pl.kernel(
        _body,
        out_type=jax.ShapeDtypeStruct((NT, KPT * B, C), jnp.float32),
        mesh=mesh,
        compiler_params=pltpu.CompilerParams(
            needs_layout_passes=False, use_tc_tiling_on_sc=True
        ),
        scratch_types=[
            pltpu.VMEM((B, 2, K), jnp.int32),
            pltpu.VMEM((KPT, L, C), jnp.float32),
            [pltpu.SemaphoreType.DMA] * KPT,
            pltpu.SemaphoreType.DMA,
        ],
    )
    return f(table, vert_flat)


def kernel(x, vertexs):
    table = x.transpose(0, 2, 3, 1).reshape(B * H * W, C)
    vert_t = vertexs.astype(jnp.int32).transpose(0, 2, 1)
    out = _sampler(table, vert_t)
    return out.reshape(K, B, C).transpose(1, 2, 0)
